# Initial kernel scaffold; baseline (speedup 1.0000x reference)
import jax
import jax.numpy as jnp
from jax.experimental import pallas as pl


def kernel(x, edge_index, W1, b1, W2, b2):
    raise NotImplementedError("write your pallas kernel here")


# speedup vs baseline: 3.8430x; 3.8430x over previous
import functools

import jax
import jax.numpy as jnp
from jax import lax
from jax.experimental import pallas as pl
from jax.experimental.pallas import tpu as pltpu
from jax.experimental.pallas import tpu_sc as plsc

N_NODES = 10000
N_EDGES = 320000
F_IN = 128
F_HID = 128
F_OUT = 64

NC = 2
NS = 16
NW = NC * NS
CHUNK = 128
KCH = 80
E_PAD = NW * KCH * CHUNK
N_PAD = 10112
ROWS_PER_SUB = N_PAD // NS
DEGW = 16

_mesh = plsc.VectorSubcoreMesh(core_axis_name="c", subcore_axis_name="s")

_SPANS = ((0, 128), (128, 128), (256, 128), (384, 128), (512, 120))


def _init_acc(zb, acc, base):
    for off, ln in _SPANS:
        pltpu.sync_copy(zb.at[pl.ds(0, ln)], acc.at[pl.ds(base + off, ln)])


def _drain_acc(acc, out_ref, c, base, buf):
    for off, ln in _SPANS:
        sl = pl.ds(base + off, ln)
        pltpu.sync_copy(acc.at[sl], buf.at[pl.ds(0, ln)])
        pltpu.sync_copy(buf.at[pl.ds(0, ln)], out_ref.at[c, sl])


@functools.partial(
    pl.kernel,
    out_type=[
        jax.ShapeDtypeStruct((NC, N_PAD, DEGW), jnp.float32),
        jax.ShapeDtypeStruct((NC, N_PAD, DEGW), jnp.float32),
    ],
    mesh=_mesh,
    compiler_params=pltpu.CompilerParams(use_tc_tiling_on_sc=False),
    scratch_types=[
        pltpu.VMEM((KCH, CHUNK), jnp.int32),
        pltpu.VMEM((KCH, CHUNK), jnp.int32),
        pltpu.VMEM((CHUNK, DEGW), jnp.float32),
        pltpu.VMEM((CHUNK, DEGW), jnp.float32),
        pltpu.VMEM_SHARED((N_PAD, DEGW), jnp.float32),
        pltpu.VMEM_SHARED((N_PAD, DEGW), jnp.float32),
    ],
)
def _sc_degrees(src_hbm, dst_hbm, ones_hbm, zeros_hbm, outs_hbm, outd_hbm,
                srcv, dstv, onesv, zb, accs, accd):
    c = lax.axis_index("c")
    s = lax.axis_index("s")
    wid = c * NS + s
    pltpu.sync_copy(src_hbm.at[wid], srcv)
    pltpu.sync_copy(dst_hbm.at[wid], dstv)
    pltpu.sync_copy(ones_hbm, onesv)
    pltpu.sync_copy(zeros_hbm, zb)
    base = s * ROWS_PER_SUB
    _init_acc(zb, accs, base)
    _init_acc(zb, accd, base)
    plsc.subcore_barrier()

    def chunk(j, carry):
        pltpu.sync_copy(onesv, accs.at[srcv.at[j]], add=True)
        pltpu.sync_copy(onesv, accd.at[dstv.at[j]], add=True)
        return carry

    lax.fori_loop(0, KCH, chunk, 0)
    plsc.subcore_barrier()
    _drain_acc(accs, outs_hbm, c, base, zb)
    _drain_acc(accd, outd_hbm, c, base, zb)


def _make_sc_agg(W):
    @functools.partial(
        pl.kernel,
        out_type=jax.ShapeDtypeStruct((NC, N_PAD, W), jnp.float32),
        mesh=_mesh,
        compiler_params=pltpu.CompilerParams(use_tc_tiling_on_sc=False),
        scratch_types=[
            pltpu.VMEM((KCH, CHUNK), jnp.int32),
            pltpu.VMEM((KCH, CHUNK), jnp.int32),
            pltpu.VMEM((CHUNK, W), jnp.float32),
            pltpu.VMEM_SHARED((N_PAD, W), jnp.float32),
            pltpu.SemaphoreType.DMA,
        ],
    )
    def _sc_agg(z_hbm, src_hbm, dst_hbm, zeros_hbm, out_hbm,
                srcv, dstv, buf, acc, sem):
        c = lax.axis_index("c")
        s = lax.axis_index("s")
        wid = c * NS + s
        pltpu.sync_copy(src_hbm.at[wid], srcv)
        pltpu.sync_copy(dst_hbm.at[wid], dstv)
        pltpu.sync_copy(zeros_hbm, buf)
        base = s * ROWS_PER_SUB
        _init_acc(buf, acc, base)
        plsc.subcore_barrier()

        def chunk(j, carry):
            pltpu.async_copy(z_hbm.at[srcv.at[j]], buf, sem).wait()
            pltpu.sync_copy(buf, acc.at[dstv.at[j]], add=True)
            return carry

        lax.fori_loop(0, KCH, chunk, 0)
        plsc.subcore_barrier()
        _drain_acc(acc, out_hbm, c, base, buf)

    return _sc_agg


_sc_agg_hid = _make_sc_agg(F_HID)
_sc_agg_out = _make_sc_agg(F_OUT)


_RB = N_PAD // 4


def _norm(p0_ref, p1_ref):
    deg = p0_ref[:, 0:1] + p1_ref[:, 0:1]
    return lax.rsqrt(jnp.maximum(deg, 1.0))


def _tc_pre_body(x_ref, ps0, ps1, w_ref, o_ref):
    ns = _norm(ps0, ps1)
    o_ref[...] = jnp.dot(x_ref[...] * ns, w_ref[...],
                         preferred_element_type=jnp.float32)


def _tc_mid_body(a0, a1, pd0, pd1, ps0, ps1, b_ref, w_ref, o_ref):
    nd = _norm(pd0, pd1)
    ns = _norm(ps0, ps1)
    h = jnp.maximum((a0[...] + a1[...]) * nd + b_ref[...], 0.0)
    o_ref[...] = jnp.dot(h * ns, w_ref[...],
                         preferred_element_type=jnp.float32)


def _tc_fin_body(a0, a1, pd0, pd1, b_ref, o_ref):
    nd = _norm(pd0, pd1)
    o_ref[...] = (a0[...] + a1[...]) * nd + b_ref[...]


def _rows(w):
    return pl.BlockSpec((_RB, w), lambda i: (i, 0))


def _full(shape):
    return pl.BlockSpec(shape, lambda i: (0, 0))


def _tc_pre(x_pad, ps0, ps1, W1):
    return pl.pallas_call(
        _tc_pre_body,
        grid=(4,),
        in_specs=[_rows(F_IN), _rows(DEGW), _rows(DEGW), _full((F_IN, F_HID))],
        out_specs=_rows(F_HID),
        out_shape=jax.ShapeDtypeStruct((N_PAD, F_HID), jnp.float32),
    )(x_pad, ps0, ps1, W1)


def _tc_mid(a0, a1, pd0, pd1, ps0, ps1, b1, W2):
    return pl.pallas_call(
        _tc_mid_body,
        grid=(4,),
        in_specs=[_rows(F_HID), _rows(F_HID), _rows(DEGW), _rows(DEGW),
                  _rows(DEGW), _rows(DEGW), _full((1, F_HID)),
                  _full((F_HID, F_OUT))],
        out_specs=_rows(F_OUT),
        out_shape=jax.ShapeDtypeStruct((N_PAD, F_OUT), jnp.float32),
    )(a0, a1, pd0, pd1, ps0, ps1, b1, W2)


def _tc_fin(a0, a1, pd0, pd1, b2):
    return pl.pallas_call(
        _tc_fin_body,
        grid=(4,),
        in_specs=[_rows(F_OUT), _rows(F_OUT), _rows(DEGW), _rows(DEGW),
                  _full((1, F_OUT))],
        out_specs=_rows(F_OUT),
        out_shape=jax.ShapeDtypeStruct((N_PAD, F_OUT), jnp.float32),
    )(a0, a1, pd0, pd1, b2)


def kernel(x, edge_index, W1, b1, W2, b2):
    src = edge_index[0].astype(jnp.int32)
    dst = edge_index[1].astype(jnp.int32)
    pad = jnp.full((E_PAD - N_EDGES,), N_NODES, jnp.int32)
    src3 = jnp.concatenate([src, pad]).reshape(NW, KCH, CHUNK)
    dst3 = jnp.concatenate([dst, pad]).reshape(NW, KCH, CHUNK)
    x_pad = jnp.concatenate(
        [x, jnp.zeros((N_PAD - N_NODES, F_IN), jnp.float32)])

    ones_deg = jnp.ones((CHUNK, DEGW), jnp.float32)
    zeros_deg = jnp.zeros((CHUNK, DEGW), jnp.float32)
    zeros_hid = jnp.zeros((CHUNK, F_HID), jnp.float32)
    zeros_out = jnp.zeros((CHUNK, F_OUT), jnp.float32)

    degs, degd = _sc_degrees(src3, dst3, ones_deg, zeros_deg)
    ps0, ps1 = degs[0], degs[1]
    pd0, pd1 = degd[0], degd[1]

    z1 = _tc_pre(x_pad, ps0, ps1, W1)
    a1 = _sc_agg_hid(z1, src3, dst3, zeros_hid)
    z2 = _tc_mid(a1[0], a1[1], pd0, pd1, ps0, ps1,
                 b1.reshape(1, F_HID), W2)
    a2 = _sc_agg_out(z2, src3, dst3, zeros_out)
    out = _tc_fin(a2[0], a2[1], pd0, pd1, b2.reshape(1, F_OUT))
    return out[:N_NODES]


# speedup vs baseline: 4.2244x; 1.0992x over previous
import functools

import jax
import jax.numpy as jnp
from jax import lax
from jax.experimental import pallas as pl
from jax.experimental.pallas import tpu as pltpu
from jax.experimental.pallas import tpu_sc as plsc

N_NODES = 10000
N_EDGES = 320000
F_IN = 128
F_HID = 128
F_OUT = 64

NC = 2
NS = 16
NW = NC * NS
CHUNK = 128
KCH = 80
E_PAD = NW * KCH * CHUNK
N_PAD = 10112
ROWS_PER_SUB = N_PAD // NS
DEGW = 16

_mesh = plsc.VectorSubcoreMesh(core_axis_name="c", subcore_axis_name="s")

_SPANS = ((0, 128), (128, 128), (256, 128), (384, 128), (512, 120))


def _init_acc(zb, acc, base):
    for off, ln in _SPANS:
        pltpu.sync_copy(zb.at[pl.ds(0, ln)], acc.at[pl.ds(base + off, ln)])


def _drain_acc(acc, out_ref, c, base, buf):
    for off, ln in _SPANS:
        sl = pl.ds(base + off, ln)
        pltpu.sync_copy(acc.at[sl], buf.at[pl.ds(0, ln)])
        pltpu.sync_copy(buf.at[pl.ds(0, ln)], out_ref.at[c, sl])


@functools.partial(
    pl.kernel,
    out_type=[
        jax.ShapeDtypeStruct((NC, N_PAD, DEGW), jnp.float32),
        jax.ShapeDtypeStruct((NC, N_PAD, DEGW), jnp.float32),
    ],
    mesh=_mesh,
    compiler_params=pltpu.CompilerParams(use_tc_tiling_on_sc=False),
    scratch_types=[
        pltpu.VMEM((KCH, CHUNK), jnp.int32),
        pltpu.VMEM((KCH, CHUNK), jnp.int32),
        pltpu.VMEM((CHUNK, DEGW), jnp.float32),
        pltpu.VMEM((CHUNK, DEGW), jnp.float32),
        pltpu.VMEM_SHARED((N_PAD, DEGW), jnp.float32),
        pltpu.VMEM_SHARED((N_PAD, DEGW), jnp.float32),
    ],
)
def _sc_degrees(src_hbm, dst_hbm, ones_hbm, zeros_hbm, outs_hbm, outd_hbm,
                srcv, dstv, onesv, zb, accs, accd):
    c = lax.axis_index("c")
    s = lax.axis_index("s")
    wid = c * NS + s
    pltpu.sync_copy(src_hbm.at[wid], srcv)
    pltpu.sync_copy(dst_hbm.at[wid], dstv)
    pltpu.sync_copy(ones_hbm, onesv)
    pltpu.sync_copy(zeros_hbm, zb)
    base = s * ROWS_PER_SUB
    _init_acc(zb, accs, base)
    _init_acc(zb, accd, base)
    plsc.subcore_barrier()

    def chunk(j, carry):
        pltpu.sync_copy(onesv, accs.at[srcv.at[j]], add=True)
        pltpu.sync_copy(onesv, accd.at[dstv.at[j]], add=True)
        return carry

    lax.fori_loop(0, KCH, chunk, 0)
    plsc.subcore_barrier()
    _drain_acc(accs, outs_hbm, c, base, zb)
    _drain_acc(accd, outd_hbm, c, base, zb)


NBUF = 2
KC2 = KCH // 2


def _make_sc_agg(W):
    @functools.partial(
        pl.kernel,
        out_type=jax.ShapeDtypeStruct((NC, N_PAD, W), jnp.float32),
        mesh=_mesh,
        compiler_params=pltpu.CompilerParams(use_tc_tiling_on_sc=False),
        scratch_types=[
            pltpu.VMEM((KC2, CHUNK), jnp.int32),
            pltpu.VMEM((KC2, CHUNK), jnp.int32),
            pltpu.VMEM((CHUNK, W), jnp.float32),
            pltpu.VMEM((CHUNK, W), jnp.float32),
            pltpu.VMEM_SHARED((N_PAD, W), jnp.float32),
            pltpu.SemaphoreType.DMA,
            pltpu.SemaphoreType.DMA,
        ],
    )
    def _sc_agg(z_hbm, src_hbm, dst_hbm, zeros_hbm, out_hbm,
                srcv, dstv, b0, b1, acc, s0, s1):
        bufs = (b0, b1)
        sems = (s0, s1)
        c = lax.axis_index("c")
        s = lax.axis_index("s")
        wid = c * NS + s
        pltpu.sync_copy(zeros_hbm, b0)
        base = s * ROWS_PER_SUB
        _init_acc(b0, acc, base)
        plsc.subcore_barrier()

        for half in range(2):
            off = half * KC2
            pltpu.sync_copy(src_hbm.at[wid, pl.ds(off, KC2)], srcv)
            pltpu.sync_copy(dst_hbm.at[wid, pl.ds(off, KC2)], dstv)
            for b in range(NBUF):
                pltpu.async_copy(z_hbm.at[srcv.at[b]], bufs[b], sems[b])

            def outer(i, carry):
                g = i * NBUF
                for b in range(NBUF):
                    j = g + b
                    pltpu.make_async_copy(
                        z_hbm.at[srcv.at[j]], bufs[b], sems[b]).wait()
                    pltpu.sync_copy(bufs[b], acc.at[dstv.at[j]], add=True)
                    pltpu.async_copy(
                        z_hbm.at[srcv.at[j + NBUF]], bufs[b], sems[b])
                return carry

            lax.fori_loop(0, KC2 // NBUF - 1, outer, 0)
            tail = KC2 - NBUF
            for b in range(NBUF):
                j = tail + b
                pltpu.make_async_copy(
                    z_hbm.at[srcv.at[j]], bufs[b], sems[b]).wait()
                pltpu.sync_copy(bufs[b], acc.at[dstv.at[j]], add=True)
        plsc.subcore_barrier()
        _drain_acc(acc, out_hbm, c, base, b0)

    return _sc_agg


_sc_agg_hid = _make_sc_agg(F_HID)
_sc_agg_out = _make_sc_agg(F_OUT)


_RB = N_PAD // 4


def _norm(p0_ref, p1_ref):
    deg = p0_ref[:, 0:1] + p1_ref[:, 0:1]
    return lax.rsqrt(jnp.maximum(deg, 1.0))


def _tc_pre_body(x_ref, ps0, ps1, w_ref, o_ref):
    ns = _norm(ps0, ps1)
    o_ref[...] = jnp.dot(x_ref[...] * ns, w_ref[...],
                         preferred_element_type=jnp.float32)


def _tc_mid_body(a0, a1, pd0, pd1, ps0, ps1, b_ref, w_ref, o_ref):
    nd = _norm(pd0, pd1)
    ns = _norm(ps0, ps1)
    h = jnp.maximum((a0[...] + a1[...]) * nd + b_ref[...], 0.0)
    o_ref[...] = jnp.dot(h * ns, w_ref[...],
                         preferred_element_type=jnp.float32)


def _tc_fin_body(a0, a1, pd0, pd1, b_ref, o_ref):
    nd = _norm(pd0, pd1)
    o_ref[...] = (a0[...] + a1[...]) * nd + b_ref[...]


def _rows(w):
    return pl.BlockSpec((_RB, w), lambda i: (i, 0))


def _full(shape):
    return pl.BlockSpec(shape, lambda i: (0, 0))


def _tc_pre(x_pad, ps0, ps1, W1):
    return pl.pallas_call(
        _tc_pre_body,
        grid=(4,),
        in_specs=[_rows(F_IN), _rows(DEGW), _rows(DEGW), _full((F_IN, F_HID))],
        out_specs=_rows(F_HID),
        out_shape=jax.ShapeDtypeStruct((N_PAD, F_HID), jnp.float32),
    )(x_pad, ps0, ps1, W1)


def _tc_mid(a0, a1, pd0, pd1, ps0, ps1, b1, W2):
    return pl.pallas_call(
        _tc_mid_body,
        grid=(4,),
        in_specs=[_rows(F_HID), _rows(F_HID), _rows(DEGW), _rows(DEGW),
                  _rows(DEGW), _rows(DEGW), _full((1, F_HID)),
                  _full((F_HID, F_OUT))],
        out_specs=_rows(F_OUT),
        out_shape=jax.ShapeDtypeStruct((N_PAD, F_OUT), jnp.float32),
    )(a0, a1, pd0, pd1, ps0, ps1, b1, W2)


def _tc_fin(a0, a1, pd0, pd1, b2):
    return pl.pallas_call(
        _tc_fin_body,
        grid=(4,),
        in_specs=[_rows(F_OUT), _rows(F_OUT), _rows(DEGW), _rows(DEGW),
                  _full((1, F_OUT))],
        out_specs=_rows(F_OUT),
        out_shape=jax.ShapeDtypeStruct((N_PAD, F_OUT), jnp.float32),
    )(a0, a1, pd0, pd1, b2)


def kernel(x, edge_index, W1, b1, W2, b2):
    src = edge_index[0].astype(jnp.int32)
    dst = edge_index[1].astype(jnp.int32)
    pad = jnp.full((E_PAD - N_EDGES,), N_NODES, jnp.int32)
    src3 = jnp.concatenate([src, pad]).reshape(NW, KCH, CHUNK)
    dst3 = jnp.concatenate([dst, pad]).reshape(NW, KCH, CHUNK)
    x_pad = jnp.concatenate(
        [x, jnp.zeros((N_PAD - N_NODES, F_IN), jnp.float32)])

    ones_deg = jnp.ones((CHUNK, DEGW), jnp.float32)
    zeros_deg = jnp.zeros((CHUNK, DEGW), jnp.float32)
    zeros_hid = jnp.zeros((CHUNK, F_HID), jnp.float32)
    zeros_out = jnp.zeros((CHUNK, F_OUT), jnp.float32)

    degs, degd = _sc_degrees(src3, dst3, ones_deg, zeros_deg)
    ps0, ps1 = degs[0], degs[1]
    pd0, pd1 = degd[0], degd[1]

    z1 = _tc_pre(x_pad, ps0, ps1, W1)
    a1 = _sc_agg_hid(z1, src3, dst3, zeros_hid)
    z2 = _tc_mid(a1[0], a1[1], pd0, pd1, ps0, ps1,
                 b1.reshape(1, F_HID), W2)
    a2 = _sc_agg_out(z2, src3, dst3, zeros_out)
    out = _tc_fin(a2[0], a2[1], pd0, pd1, b2.reshape(1, F_OUT))
    return out[:N_NODES]
